# baseline (speedup 1.0000x reference)
import functools

import jax
import jax.numpy as jnp
from jax import lax
from jax.experimental import pallas as pl
from jax.experimental.pallas import tpu as pltpu
from jax.experimental.pallas import tpu_sc as plsc

N_NODES = 10000
NP = 10240
D = 64
D_IN = 128
N_EDGES = 320000
E_TOT = N_EDGES + N_NODES
NC, NS, L = 2, 16, 16
NW = NC * NS
CHUNK = 128
KC = CHUNK // 128
C_PER_W = 10752
EP = C_PER_W * NW
NCHUNK = C_PER_W // CHUNK
IDXR = C_PER_W // 128
ROWS_PER_TILE = NP // NS
NEG = 0.2


def _dg(x, w):
    return lax.dot_general(x, w, (((1,), (1,)), ((), ())),
                           preferred_element_type=jnp.float32)


def _mm1_body(x_ref, wl_ref, wr_ref, a_ref, q_ref, p_ref):
    xb = x_ref[...]
    av = a_ref[...]
    q_ref[...] = _dg(xb, wl_ref[...]) * av
    p_ref[...] = _dg(xb, wr_ref[...]) * av


def _mm1(xp, wl, wr, a):
    grid = NP // 1024
    return pl.pallas_call(
        _mm1_body,
        grid=(grid,),
        in_specs=[
            pl.BlockSpec((1024, D_IN), lambda i: (i, 0)),
            pl.BlockSpec((D, D_IN), lambda i: (0, 0)),
            pl.BlockSpec((D, D_IN), lambda i: (0, 0)),
            pl.BlockSpec((1, D), lambda i: (0, 0)),
        ],
        out_specs=[
            pl.BlockSpec((1024, D), lambda i: (i, 0)),
            pl.BlockSpec((1024, D), lambda i: (i, 0)),
        ],
        out_shape=[
            jax.ShapeDtypeStruct((NP, D), jnp.float32),
            jax.ShapeDtypeStruct((NP, D), jnp.float32),
        ],
    )(xp, wl, wr, a)


def _epi(acc_ref, den_ref, ap_ref, b_ref):
    accs = acc_ref[0] + acc_ref[1]
    dens = den_ref[0] + den_ref[1]
    h = accs / ap_ref[...] / (dens + 1e-16) + b_ref[...]
    return jnp.maximum(h, 0.0)


def _epi_mm_body(acc_ref, den_ref, ap_ref, b_ref, wl_ref, wr_ref, a_ref,
                 q_ref, p_ref):
    h = _epi(acc_ref, den_ref, ap_ref, b_ref)
    av = a_ref[...]
    q_ref[...] = _dg(h, wl_ref[...]) * av
    p_ref[...] = _dg(h, wr_ref[...]) * av


def _epi_mm(acc, den, a_prev, b, wl, wr, a):
    grid = NP // 1024
    return pl.pallas_call(
        _epi_mm_body,
        grid=(grid,),
        in_specs=[
            pl.BlockSpec((NC, 1024, D), lambda i: (0, i, 0)),
            pl.BlockSpec((NC, 1024, 1), lambda i: (0, i, 0)),
            pl.BlockSpec((1, D), lambda i: (0, 0)),
            pl.BlockSpec((1, D), lambda i: (0, 0)),
            pl.BlockSpec((D, D), lambda i: (0, 0)),
            pl.BlockSpec((D, D), lambda i: (0, 0)),
            pl.BlockSpec((1, D), lambda i: (0, 0)),
        ],
        out_specs=[
            pl.BlockSpec((1024, D), lambda i: (i, 0)),
            pl.BlockSpec((1024, D), lambda i: (i, 0)),
        ],
        out_shape=[
            jax.ShapeDtypeStruct((NP, D), jnp.float32),
            jax.ShapeDtypeStruct((NP, D), jnp.float32),
        ],
    )(acc, den, a_prev, b, wl, wr, a)


def _epi_only_body(acc_ref, den_ref, ap_ref, b_ref, h_ref):
    h_ref[...] = _epi(acc_ref, den_ref, ap_ref, b_ref)


def _epi_only(acc, den, a_prev, b):
    grid = NP // 1024
    return pl.pallas_call(
        _epi_only_body,
        grid=(grid,),
        in_specs=[
            pl.BlockSpec((NC, 1024, D), lambda i: (0, i, 0)),
            pl.BlockSpec((NC, 1024, 1), lambda i: (0, i, 0)),
            pl.BlockSpec((1, D), lambda i: (0, 0)),
            pl.BlockSpec((1, D), lambda i: (0, 0)),
        ],
        out_specs=pl.BlockSpec((1024, D), lambda i: (i, 0)),
        out_shape=jax.ShapeDtypeStruct((NP, D), jnp.float32),
    )(acc, den, a_prev, b)


def _sc_edge_body(pk_hbm, p_hbm, q_hbm, coef_hbm,
                  acc_out, den_out,
                  pk_all,
                  p_a, q_a, ex_a, iib_a, jjb_a,
                  p_b, q_b, ex_b, iib_b, jjb_b, coef_v,
                  q_tab, acc_s, den_s,
                  gsem_a, gsem_b):
    cid = lax.axis_index("c")
    sid = lax.axis_index("s")
    wid = sid * NC + cid

    zero = jnp.zeros((L,), jnp.float32)

    pltpu.sync_copy(pk_hbm.at[wid], pk_all)
    pltpu.sync_copy(coef_hbm, coef_v)
    base_r = sid * ROWS_PER_TILE
    pltpu.sync_copy(q_hbm.at[pl.ds(base_r, ROWS_PER_TILE)],
                    q_tab.at[pl.ds(base_r, ROWS_PER_TILE)])

    def zrow(r, _):
        for k in range(D // L):
            p_a[r, k * L:(k + 1) * L] = zero
        return 0
    lax.fori_loop(0, 128, zrow, 0)

    def zex(r, _):
        ex_a[pl.ds(r * L, L)] = zero
        return 0
    lax.fori_loop(0, 128 // L, zex, 0)

    for m in range(ROWS_PER_TILE // 128):
        pltpu.sync_copy(p_a.at[pl.ds(0, 128)],
                        acc_s.at[pl.ds(base_r + m * 128, 128)])
        pltpu.sync_copy(ex_a.at[pl.ds(0, 128)],
                        den_s.at[pl.ds(base_r + m * 128, 128)])

    cvec = [0.4 * jnp.sign(coef_v[pl.ds(k * L, L)]) for k in range(D // L)]
    lane = lax.iota(jnp.int32, L)
    perms = [lane ^ sh for sh in (8, 4, 2, 1)]

    gdn = lax.GatherDimensionNumbers(
        offset_dims=(), collapsed_slice_dims=(0,), start_index_map=(0,))

    def hsum_bcast(v):
        for pidx in perms:
            v = v + lax.gather(v, pidx[:, None], gdn, slice_sizes=(1,),
                               mode=lax.GatherScatterMode.PROMISE_IN_BOUNDS)
        return v

    plsc.subcore_barrier()

    bufs = [(p_a, q_a, ex_a, iib_a, jjb_a, gsem_a),
            (p_b, q_b, ex_b, iib_b, jjb_b, gsem_b)]

    def fetch(t, b):
        p_buf, q_buf, _, iib, jjb, gsem = bufs[b]
        for k in range(128 // L):
            pk = pk_all[t, k * L:(k + 1) * L]
            iib[0, k * L:(k + 1) * L] = lax.shift_right_logical(pk, 16)
            jjb[0, k * L:(k + 1) * L] = jnp.bitwise_and(pk, 0xFFFF)
        pltpu.async_copy(p_hbm.at[iib.at[0]], p_buf, gsem)
        pltpu.async_copy(q_tab.at[jjb.at[0]], q_buf, gsem)

    def gdrain(b):
        p_buf, q_buf, _, iib, jjb, gsem = bufs[b]
        pltpu.make_async_copy(p_hbm.at[iib.at[0]], p_buf, gsem).wait()
        pltpu.make_async_copy(q_tab.at[jjb.at[0]], q_buf, gsem).wait()

    def compute(b):
        p_buf, q_buf, ex_buf = bufs[b][:3]

        def blk_body(eb, _):
            exv = zero
            for e in range(L):
                r = eb * L + e
                qs = [q_buf[r, k * L:(k + 1) * L] for k in range(D // L)]
                s1 = zero
                s2 = zero
                for k in range(D // L):
                    v = p_buf[r, k * L:(k + 1) * L] + qs[k]
                    s1 = s1 + v
                    s2 = s2 + cvec[k] * jnp.abs(v)
                es = jnp.exp(hsum_bcast((0.5 + 0.5 * NEG) * s1 + s2))
                exv = jnp.where(lane == e, es, exv)
                for k in range(D // L):
                    q_buf[r, k * L:(k + 1) * L] = qs[k] * es
            ex_buf[pl.ds(eb * L, L)] = exv
            return 0
        lax.fori_loop(0, CHUNK // L, blk_body, 0)

    def scatter(b):
        _, q_buf, ex_buf, iib, _, _ = bufs[b]
        pltpu.sync_copy(q_buf, acc_s.at[iib.at[0]], add=True)
        pltpu.sync_copy(ex_buf, den_s.at[iib.at[0]], add=True)

    fetch(0, 0)

    def pair_body(t2, _):
        ta = 2 * t2
        fetch(ta + 1, 1)
        gdrain(0)
        compute(0)
        scatter(0)
        fetch(jnp.minimum(ta + 2, NCHUNK - 1), 0)
        gdrain(1)
        compute(1)
        scatter(1)
        return 0

    lax.fori_loop(0, NCHUNK // 2, pair_body, 0)
    gdrain(0)

    plsc.subcore_barrier()

    pltpu.sync_copy(acc_s.at[pl.ds(base_r, ROWS_PER_TILE)],
                    acc_out.at[cid, pl.ds(base_r, ROWS_PER_TILE)])
    pltpu.sync_copy(den_s.at[pl.ds(base_r, ROWS_PER_TILE)],
                    den_out.at[cid, pl.ds(base_r, ROWS_PER_TILE)])


_sc_edge = functools.partial(
    pl.kernel,
    out_type=[
        jax.ShapeDtypeStruct((NC, NP, D), jnp.float32),
        jax.ShapeDtypeStruct((NC, NP), jnp.float32),
    ],
    mesh=plsc.VectorSubcoreMesh(
        core_axis_name="c", subcore_axis_name="s",
        num_cores=NC, num_subcores=NS),
    compiler_params=pltpu.CompilerParams(use_tc_tiling_on_sc=False),
    scratch_types=[
        pltpu.VMEM((IDXR, 128), jnp.int32),
        pltpu.VMEM((CHUNK, D), jnp.float32),
        pltpu.VMEM((CHUNK, D), jnp.float32),
        pltpu.VMEM((CHUNK,), jnp.float32),
        pltpu.VMEM((1, 128), jnp.int32),
        pltpu.VMEM((1, 128), jnp.int32),
        pltpu.VMEM((CHUNK, D), jnp.float32),
        pltpu.VMEM((CHUNK, D), jnp.float32),
        pltpu.VMEM((CHUNK,), jnp.float32),
        pltpu.VMEM((1, 128), jnp.int32),
        pltpu.VMEM((1, 128), jnp.int32),
        pltpu.VMEM((D,), jnp.float32),
        pltpu.VMEM_SHARED((NP, D), jnp.float32),
        pltpu.VMEM_SHARED((NP, D), jnp.float32),
        pltpu.VMEM_SHARED((NP,), jnp.float32),
        pltpu.SemaphoreType.DMA,
        pltpu.SemaphoreType.DMA,
    ],
)(_sc_edge_body)


def kernel(x, edge_index, edge_attr,
           W1l, W1r, a1, b1, W2l, W2r, a2, b2, W3l, W3r, a3, b3):
    del edge_attr
    f32 = jnp.float32
    loop = jnp.arange(N_NODES, dtype=jnp.int32)
    pad = EP - E_TOT
    padi = N_NODES + (jnp.arange(pad, dtype=jnp.int32) % (NP - N_NODES))
    ii = jnp.concatenate([edge_index[0], loop, padi])
    jj = jnp.concatenate([edge_index[1], loop, jnp.zeros((pad,), jnp.int32)])
    pk = ((ii << 16) | jj).reshape(NW, IDXR, 128)
    xp = jnp.pad(x, ((0, NP - N_NODES), (0, 0)))

    a1r = a1.reshape(1, D)
    a2r = a2.reshape(1, D)
    a3r = a3.reshape(1, D)
    b1r = b1.reshape(1, D)
    b2r = b2.reshape(1, D)
    b3r = b3.reshape(1, D)

    q1, p1 = _mm1(xp, W1l, W1r, a1r)
    acc1, den1 = _sc_edge(pk, p1, q1, a1)
    q2, p2 = _epi_mm(acc1, den1.reshape(NC, NP, 1), a1r, b1r, W2l, W2r, a2r)
    acc2, den2 = _sc_edge(pk, p2, q2, a2)
    q3, p3 = _epi_mm(acc2, den2.reshape(NC, NP, 1), a2r, b2r, W3l, W3r, a3r)
    acc3, den3 = _sc_edge(pk, p3, q3, a3)
    h3 = _epi_only(acc3, den3.reshape(NC, NP, 1), a3r, b3r)
    return h3[:N_NODES]


# speedup vs baseline: 1.0002x; 1.0002x over previous
import functools

import jax
import jax.numpy as jnp
from jax import lax
from jax.experimental import pallas as pl
from jax.experimental.pallas import tpu as pltpu
from jax.experimental.pallas import tpu_sc as plsc

N_NODES = 10000
NP = 10240
D = 64
D_IN = 128
N_EDGES = 320000
E_TOT = N_EDGES + N_NODES
NC, NS, L = 2, 16, 16
NW = NC * NS
CHUNK = 128
KC = CHUNK // 128
C_PER_W = 10752
EP = C_PER_W * NW
NCHUNK = C_PER_W // CHUNK
IDXR = C_PER_W // 128
ROWS_PER_TILE = NP // NS
NEG = 0.2


def _dg(x, w):
    return lax.dot_general(x, w, (((1,), (1,)), ((), ())),
                           preferred_element_type=jnp.float32)


def _mm1_body(x_ref, wl_ref, wr_ref, a_ref, q_ref, p_ref):
    xb = x_ref[...]
    av = a_ref[...]
    q_ref[...] = _dg(xb, wl_ref[...]) * av
    p_ref[...] = _dg(xb, wr_ref[...]) * av


def _mm1(xp, wl, wr, a):
    grid = NP // 1024
    return pl.pallas_call(
        _mm1_body,
        grid=(grid,),
        in_specs=[
            pl.BlockSpec((1024, D_IN), lambda i: (i, 0)),
            pl.BlockSpec((D, D_IN), lambda i: (0, 0)),
            pl.BlockSpec((D, D_IN), lambda i: (0, 0)),
            pl.BlockSpec((1, D), lambda i: (0, 0)),
        ],
        out_specs=[
            pl.BlockSpec((1024, D), lambda i: (i, 0)),
            pl.BlockSpec((1024, D), lambda i: (i, 0)),
        ],
        out_shape=[
            jax.ShapeDtypeStruct((NP, D), jnp.float32),
            jax.ShapeDtypeStruct((NP, D), jnp.float32),
        ],
    )(xp, wl, wr, a)


def _epi(acc_ref, den_ref, ap_ref, b_ref):
    accs = acc_ref[0] + acc_ref[1]
    dens = den_ref[0] + den_ref[1]
    h = accs / ap_ref[...] / (dens + 1e-16) + b_ref[...]
    return jnp.maximum(h, 0.0)


def _epi_mm_body(acc_ref, den_ref, ap_ref, b_ref, wl_ref, wr_ref, a_ref,
                 q_ref, p_ref):
    h = _epi(acc_ref, den_ref, ap_ref, b_ref)
    av = a_ref[...]
    q_ref[...] = _dg(h, wl_ref[...]) * av
    p_ref[...] = _dg(h, wr_ref[...]) * av


def _epi_mm(acc, den, a_prev, b, wl, wr, a):
    grid = NP // 1024
    return pl.pallas_call(
        _epi_mm_body,
        grid=(grid,),
        in_specs=[
            pl.BlockSpec((NC, 1024, D), lambda i: (0, i, 0)),
            pl.BlockSpec((NC, 1024, 1), lambda i: (0, i, 0)),
            pl.BlockSpec((1, D), lambda i: (0, 0)),
            pl.BlockSpec((1, D), lambda i: (0, 0)),
            pl.BlockSpec((D, D), lambda i: (0, 0)),
            pl.BlockSpec((D, D), lambda i: (0, 0)),
            pl.BlockSpec((1, D), lambda i: (0, 0)),
        ],
        out_specs=[
            pl.BlockSpec((1024, D), lambda i: (i, 0)),
            pl.BlockSpec((1024, D), lambda i: (i, 0)),
        ],
        out_shape=[
            jax.ShapeDtypeStruct((NP, D), jnp.float32),
            jax.ShapeDtypeStruct((NP, D), jnp.float32),
        ],
    )(acc, den, a_prev, b, wl, wr, a)


def _epi_only_body(acc_ref, den_ref, ap_ref, b_ref, h_ref):
    h_ref[...] = _epi(acc_ref, den_ref, ap_ref, b_ref)


def _epi_only(acc, den, a_prev, b):
    grid = NP // 1024
    return pl.pallas_call(
        _epi_only_body,
        grid=(grid,),
        in_specs=[
            pl.BlockSpec((NC, 1024, D), lambda i: (0, i, 0)),
            pl.BlockSpec((NC, 1024, 1), lambda i: (0, i, 0)),
            pl.BlockSpec((1, D), lambda i: (0, 0)),
            pl.BlockSpec((1, D), lambda i: (0, 0)),
        ],
        out_specs=pl.BlockSpec((1024, D), lambda i: (i, 0)),
        out_shape=jax.ShapeDtypeStruct((NP, D), jnp.float32),
    )(acc, den, a_prev, b)


def _sc_edge_body(ii_hbm, jj_hbm, p_hbm, q_hbm, coef_hbm,
                  acc_out, den_out,
                  ii_all, jj_all,
                  p_a, q_a, ex_a, p_b, q_b, ex_b, p_c, q_c, ex_c, coef_v,
                  acc_s, den_s,
                  gsem_a, gsem_b, gsem_c, ssem_a, ssem_b, ssem_c):
    cid = lax.axis_index("c")
    sid = lax.axis_index("s")
    wid = sid * NC + cid

    zero = jnp.zeros((L,), jnp.float32)

    pltpu.sync_copy(ii_hbm.at[wid], ii_all)
    pltpu.sync_copy(jj_hbm.at[wid], jj_all)
    pltpu.sync_copy(coef_hbm, coef_v)

    def zrow(r, _):
        for k in range(D // L):
            p_a[r, k * L:(k + 1) * L] = zero
        return 0
    lax.fori_loop(0, 128, zrow, 0)

    def zex(r, _):
        ex_a[pl.ds(r * L, L)] = zero
        return 0
    lax.fori_loop(0, 128 // L, zex, 0)

    base_r = sid * ROWS_PER_TILE
    for m in range(ROWS_PER_TILE // 128):
        pltpu.sync_copy(p_a.at[pl.ds(0, 128)],
                        acc_s.at[pl.ds(base_r + m * 128, 128)])
        pltpu.sync_copy(ex_a.at[pl.ds(0, 128)],
                        den_s.at[pl.ds(base_r + m * 128, 128)])

    cvec = [0.4 * jnp.sign(coef_v[pl.ds(k * L, L)]) for k in range(D // L)]
    lane = lax.iota(jnp.int32, L)
    perms = [lane ^ sh for sh in (8, 4, 2, 1)]

    gdn = lax.GatherDimensionNumbers(
        offset_dims=(), collapsed_slice_dims=(0,), start_index_map=(0,))

    def hsum_bcast(v):
        for pidx in perms:
            v = v + lax.gather(v, pidx[:, None], gdn, slice_sizes=(1,),
                               mode=lax.GatherScatterMode.PROMISE_IN_BOUNDS)
        return v

    plsc.subcore_barrier()

    bufs = [(p_a, q_a, ex_a, gsem_a, ssem_a),
            (p_b, q_b, ex_b, gsem_b, ssem_b),
            (p_c, q_c, ex_c, gsem_c, ssem_c)]

    def fetch(t, b):
        p_buf, q_buf, _, gsem, _ = bufs[b]
        for j in range(KC):
            r = t * KC + j
            pltpu.async_copy(p_hbm.at[ii_all.at[r]],
                             p_buf.at[pl.ds(j * 128, 128)], gsem)
            pltpu.async_copy(q_hbm.at[jj_all.at[r]],
                             q_buf.at[pl.ds(j * 128, 128)], gsem)

    def gdrain(b):
        p_buf, q_buf, _, gsem, _ = bufs[b]
        for j in range(KC):
            pltpu.make_async_copy(p_hbm.at[ii_all.at[0]],
                                  p_buf.at[pl.ds(j * 128, 128)], gsem).wait()
            pltpu.make_async_copy(q_hbm.at[jj_all.at[0]],
                                  q_buf.at[pl.ds(j * 128, 128)], gsem).wait()

    def compute(b):
        p_buf, q_buf, ex_buf, _, _ = bufs[b]

        def blk_body(eb, _):
            exv = zero
            for e in range(L):
                r = eb * L + e
                qs = [q_buf[r, k * L:(k + 1) * L] for k in range(D // L)]
                s1 = zero
                s2 = zero
                for k in range(D // L):
                    v = p_buf[r, k * L:(k + 1) * L] + qs[k]
                    s1 = s1 + v
                    s2 = s2 + cvec[k] * jnp.abs(v)
                es = jnp.exp(hsum_bcast((0.5 + 0.5 * NEG) * s1 + s2))
                exv = jnp.where(lane == e, es, exv)
                for k in range(D // L):
                    q_buf[r, k * L:(k + 1) * L] = qs[k] * es
            ex_buf[pl.ds(eb * L, L)] = exv
            return 0
        lax.fori_loop(0, CHUNK // L, blk_body, 0)

    def scatter(t, b):
        _, q_buf, ex_buf, _, ssem = bufs[b]
        for j in range(KC):
            r = t * KC + j
            pltpu.async_copy(q_buf.at[pl.ds(j * 128, 128)],
                             acc_s.at[ii_all.at[r]], ssem, add=True)
            pltpu.async_copy(ex_buf.at[pl.ds(j * 128, 128)],
                             den_s.at[ii_all.at[r]], ssem, add=True)

    def sdrain(b):
        _, q_buf, ex_buf, _, ssem = bufs[b]
        for j in range(KC):
            pltpu.make_async_copy(q_buf.at[pl.ds(j * 128, 128)],
                                  acc_s.at[ii_all.at[0]], ssem).wait()
            pltpu.make_async_copy(ex_buf.at[pl.ds(j * 128, 128)],
                                  den_s.at[ii_all.at[0]], ssem).wait()

    fetch(0, 0)
    fetch(1, 1)

    def tri_body(t3, _):
        for u in range(3):
            t = 3 * t3 + u
            b2 = (u + 2) % 3
            gdrain(u)
            compute(u)
            scatter(t, u)
            if u > 0:
                sdrain(b2)
            else:
                @pl.when(t3 > 0)
                def _():
                    sdrain(b2)
            tn = jnp.minimum(t + 2, NCHUNK - 1)
            fetch(tn, b2)
        return 0

    lax.fori_loop(0, NCHUNK // 3, tri_body, 0)
    sdrain((NCHUNK - 1) % 3)
    gdrain(0)
    gdrain(1)

    plsc.subcore_barrier()

    pltpu.sync_copy(acc_s.at[pl.ds(base_r, ROWS_PER_TILE)],
                    acc_out.at[cid, pl.ds(base_r, ROWS_PER_TILE)])
    pltpu.sync_copy(den_s.at[pl.ds(base_r, ROWS_PER_TILE)],
                    den_out.at[cid, pl.ds(base_r, ROWS_PER_TILE)])


_sc_edge = functools.partial(
    pl.kernel,
    out_type=[
        jax.ShapeDtypeStruct((NC, NP, D), jnp.float32),
        jax.ShapeDtypeStruct((NC, NP), jnp.float32),
    ],
    mesh=plsc.VectorSubcoreMesh(
        core_axis_name="c", subcore_axis_name="s",
        num_cores=NC, num_subcores=NS),
    compiler_params=pltpu.CompilerParams(use_tc_tiling_on_sc=False),
    scratch_types=[
        pltpu.VMEM((IDXR, 128), jnp.int32),
        pltpu.VMEM((IDXR, 128), jnp.int32),
        pltpu.VMEM((CHUNK, D), jnp.float32),
        pltpu.VMEM((CHUNK, D), jnp.float32),
        pltpu.VMEM((CHUNK,), jnp.float32),
        pltpu.VMEM((CHUNK, D), jnp.float32),
        pltpu.VMEM((CHUNK, D), jnp.float32),
        pltpu.VMEM((CHUNK,), jnp.float32),
        pltpu.VMEM((CHUNK, D), jnp.float32),
        pltpu.VMEM((CHUNK, D), jnp.float32),
        pltpu.VMEM((CHUNK,), jnp.float32),
        pltpu.VMEM((D,), jnp.float32),
        pltpu.VMEM_SHARED((NP, D), jnp.float32),
        pltpu.VMEM_SHARED((NP,), jnp.float32),
        pltpu.SemaphoreType.DMA,
        pltpu.SemaphoreType.DMA,
        pltpu.SemaphoreType.DMA,
        pltpu.SemaphoreType.DMA,
        pltpu.SemaphoreType.DMA,
        pltpu.SemaphoreType.DMA,
    ],
)(_sc_edge_body)


def kernel(x, edge_index, edge_attr,
           W1l, W1r, a1, b1, W2l, W2r, a2, b2, W3l, W3r, a3, b3):
    del edge_attr
    f32 = jnp.float32
    loop = jnp.arange(N_NODES, dtype=jnp.int32)
    pad = EP - E_TOT
    padi = N_NODES + (jnp.arange(pad, dtype=jnp.int32) % (NP - N_NODES))
    ii = jnp.concatenate([edge_index[0], loop, padi]).reshape(NW, IDXR, 128)
    jj = jnp.concatenate([edge_index[1], loop,
                          jnp.zeros((pad,), jnp.int32)]).reshape(NW, IDXR, 128)
    xp = jnp.pad(x, ((0, NP - N_NODES), (0, 0)))

    a1r = a1.reshape(1, D)
    a2r = a2.reshape(1, D)
    a3r = a3.reshape(1, D)
    b1r = b1.reshape(1, D)
    b2r = b2.reshape(1, D)
    b3r = b3.reshape(1, D)

    q1, p1 = _mm1(xp, W1l, W1r, a1r)
    acc1, den1 = _sc_edge(ii, jj, p1, q1, a1)
    q2, p2 = _epi_mm(acc1, den1.reshape(NC, NP, 1), a1r, b1r, W2l, W2r, a2r)
    acc2, den2 = _sc_edge(ii, jj, p2, q2, a2)
    q3, p3 = _epi_mm(acc2, den2.reshape(NC, NP, 1), a2r, b2r, W3l, W3r, a3r)
    acc3, den3 = _sc_edge(ii, jj, p3, q3, a3)
    h3 = _epi_only(acc3, den3.reshape(NC, NP, 1), a3r, b3r)
    return h3[:N_NODES]
